# baseline (speedup 1.0000x reference)
import functools

import numpy as np
import jax
import jax.numpy as jnp
from jax import lax
from jax.experimental import pallas as pl
from jax.experimental.pallas import tpu as pltpu
from jax.experimental.pallas import tpu_sc as plsc

N = 10000
E = 320000
H = 128
NG = 50
CUTOFF = 6.0
NGRAPH = 128
GR = 136
NPAD = 10240
NC, NS = 2, 16
NW = NC * NS
EW = E // NW
CCH = 80
NCH_E = EW // CCH
PWN = NPAD // NW
NCH_N = PWN // CCH
RT = NPAD // NS
DELTA = CUTOFF / (NG - 1)
COEFF = -0.5 / DELTA**2
BE = 1280
BN = 1024

_f32 = jnp.float32


def _mesh():
    return plsc.VectorSubcoreMesh(core_axis_name="c", subcore_axis_name="s",
                                  num_cores=NC, num_subcores=NS)


def _wid():
    return lax.axis_index("s") * NC + lax.axis_index("c")


def _sc_gather_rows(table, idx, d, nch, per_w):
    ntot = idx.shape[0]

    @functools.partial(
        pl.kernel,
        out_type=jax.ShapeDtypeStruct((ntot, d), _f32),
        mesh=_mesh(),
        scratch_types=[pltpu.VMEM((CCH,), jnp.int32),
                       pltpu.VMEM((CCH, d), _f32),
                       pltpu.SemaphoreType.DMA],
    )
    def k(tab_h, idx_h, out_h, idx_v, row_v, sem):
        base0 = _wid() * per_w

        def body(kk, c):
            b = pl.multiple_of(base0 + kk * CCH, 8)
            pltpu.sync_copy(idx_h.at[pl.ds(b, CCH)], idx_v)
            pltpu.async_copy(tab_h.at[idx_v], row_v, sem).wait()
            pltpu.sync_copy(row_v, out_h.at[pl.ds(b, CCH)])
            return c

        lax.fori_loop(0, nch, body, 0)

    return k(table, idx)


def _sc_rel(posp, src3, dst3):
    @functools.partial(
        pl.kernel,
        out_type=jax.ShapeDtypeStruct((E, 16), _f32),
        mesh=_mesh(),
        scratch_types=[pltpu.VMEM((NCH_E, CCH), jnp.int32),
                       pltpu.VMEM((NCH_E, CCH), jnp.int32),
                       pltpu.VMEM((CCH, 16), _f32),
                       pltpu.VMEM((CCH, 16), _f32),
                       pltpu.VMEM((CCH, 16), _f32),
                       pltpu.VMEM((CCH, 16), _f32),
                       pltpu.SemaphoreType.DMA,
                       pltpu.SemaphoreType.DMA,
                       pltpu.SemaphoreType.DMA,
                       pltpu.SemaphoreType.DMA],
        compiler_params=pltpu.CompilerParams(use_tc_tiling_on_sc=False),
    )
    def k(pos_h, src_h, dst_h, out_h, si_v, di_v, a0, b0, a1, b1,
          sg0, sl0, sg1, sl1):
        w = _wid()
        base0 = w * EW
        bufs = ((a0, b0, sg0, sl0), (a1, b1, sg1, sl1))
        pltpu.sync_copy(src_h.at[w], si_v)
        pltpu.sync_copy(dst_h.at[w], di_v)

        def start(kk, b):
            av, bv, sg, sl = bufs[b]
            pltpu.async_copy(pos_h.at[si_v.at[kk]], av, sg)
            pltpu.async_copy(pos_h.at[di_v.at[kk]], bv, sl)

        def fin(kk, b):
            av, bv, sg, sl = bufs[b]
            pltpu.make_async_copy(pos_h.at[si_v.at[kk]], av, sg).wait()
            pltpu.make_async_copy(pos_h.at[di_v.at[kk]], bv, sl).wait()

            def row(r, cc):
                bv[r, :] = bv[r, :] - av[r, :]
                return cc

            lax.fori_loop(0, CCH, row, 0)
            off = pl.multiple_of(base0 + kk * CCH, 8)
            pltpu.sync_copy(bv, out_h.at[pl.ds(off, CCH)])

        start(0, 0)
        start(1, 1)

        def body(j, c):
            kk = j * 2
            fin(kk, 0)
            start(kk + 2, 0)
            fin(kk + 1, 1)

            @pl.when(kk + 3 < NCH_E)
            def _():
                start(kk + 3, 1)

            return c

        lax.fori_loop(0, NCH_E // 2, body, 0)
        fin(NCH_E - 1, 0)

    return k(posp, src3, dst3)


MC = 40
MNCH = EW // MC


def _sc_msg(p, eproj, src, dst, zeros_n):
    @functools.partial(
        pl.kernel,
        out_type=jax.ShapeDtypeStruct((NC, NPAD, H), _f32),
        mesh=_mesh(),
        scratch_types=[pltpu.VMEM((MC,), jnp.int32),
                       pltpu.VMEM((MC,), jnp.int32),
                       pltpu.VMEM((MC, H), _f32),
                       pltpu.VMEM((MC, H // 2), _f32),
                       pltpu.VMEM((MC, H), _f32),
                       pltpu.VMEM((MC, H // 2), _f32),
                       pltpu.VMEM((MC,), jnp.int32),
                       pltpu.VMEM((MC,), jnp.int32),
                       pltpu.VMEM((MC,), jnp.int32),
                       pltpu.VMEM((MC,), jnp.int32),
                       pltpu.VMEM((MC, H), _f32),
                       pltpu.VMEM((MC, H), _f32),
                       pltpu.VMEM((MC, H), _f32),
                       pltpu.VMEM((MC, H), _f32),
                       pltpu.VMEM_SHARED((NPAD, H), _f32),
                       pltpu.SemaphoreType.DMA,
                       pltpu.SemaphoreType.DMA,
                       pltpu.SemaphoreType.DMA,
                       pltpu.SemaphoreType.DMA,
                       pltpu.SemaphoreType.DMA,
                       pltpu.SemaphoreType.DMA,
                       pltpu.SemaphoreType.DMA,
                       pltpu.SemaphoreType.DMA,
                       pltpu.SemaphoreType.DMA,
                       pltpu.SemaphoreType.DMA,
                       pltpu.SemaphoreType.DMA,
                       pltpu.SemaphoreType.DMA,
                       pltpu.SemaphoreType.DMA,
                       pltpu.SemaphoreType.DMA],
        compiler_params=pltpu.CompilerParams(needs_layout_passes=False),
    )
    def k(p_h, ep_h, src_h, dst_h, z_h, out_h,
          si0, si1, a0, b0, a1, b1,
          di0, di1, di2, di3, m0, m1, m2, m3, agg_s,
          ssi0, ssi1, sg0, sl0, sg1, sl1,
          sdi0, sdi1, sdi2, sdi3, ss0, ss1, ss2, ss3):
        cid = lax.axis_index("c")
        sid = lax.axis_index("s")
        w = sid * NC + cid
        base0 = w * MC * MNCH
        pltpu.sync_copy(z_h.at[pl.ds(sid * RT, RT)],
                        agg_s.at[pl.ds(sid * RT, RT)])
        data = ((si0, a0, b0, ssi0, sg0, sl0),
                (si1, a1, b1, ssi1, sg1, sl1))
        ring = ((di0, m0, sdi0, ss0), (di1, m1, sdi1, ss1),
                (di2, m2, sdi2, ss2), (di3, m3, sdi3, ss3))

        def sslice(h_ref, kk):
            return h_ref.at[pl.ds(pl.multiple_of(base0 + kk * MC, 8), MC)]

        def start(kk, d):
            si, av, bv, ssi, sg, sl = data[d]
            pltpu.make_async_copy(sslice(src_h, kk), si, ssi).wait()
            pltpu.async_copy(p_h.at[si], av, sg)
            pltpu.async_copy(sslice(ep_h, kk), bv, sl)

        def fin(kk, d, q):
            si, av, bv, ssi, sg, sl = data[d]
            di, mv, sdi, ss = ring[q]
            pltpu.make_async_copy(p_h.at[si], av, sg).wait()
            pltpu.make_async_copy(sslice(ep_h, kk), bv, sl).wait()

            @pl.when(kk + 2 < MNCH)
            def _():
                pltpu.async_copy(sslice(src_h, kk + 2), si, ssi)

            @pl.when(kk >= 4)
            def _():
                pltpu.make_async_copy(mv, agg_s.at[di], ss).wait()

            pltpu.async_copy(sslice(dst_h, kk), di, sdi)

            def row(r, cc):
                for j in range(H // 32):
                    pk = plsc.bitcast(bv[r, pl.ds(j * 16, 16)], jnp.bfloat16)
                    x0, x1 = plsc.unpack(pk,
                                         format=plsc.PackFormat.INTERLEAVED)
                    v0 = av[r, pl.ds(j * 32, 16)] + x0
                    v1 = av[r, pl.ds(j * 32 + 16, 16)] + x1
                    mv[r, pl.ds(j * 32, 16)] = v0 / (1.0 + jnp.exp(-v0))
                    mv[r, pl.ds(j * 32 + 16, 16)] = v1 / (1.0 + jnp.exp(-v1))
                return cc

            lax.fori_loop(0, MC, row, 0)
            pltpu.make_async_copy(sslice(dst_h, kk), di, sdi).wait()
            pltpu.async_copy(mv, agg_s.at[di], ss, add=True)

        pltpu.async_copy(sslice(src_h, 0), si0, ssi0)
        pltpu.async_copy(sslice(src_h, 1), si1, ssi1)
        plsc.subcore_barrier()
        start(0, 0)
        start(1, 1)

        def body(j, c):
            kk = j * 4
            fin(kk, 0, 0)
            start(kk + 2, 0)
            fin(kk + 1, 1, 1)
            start(kk + 3, 1)
            fin(kk + 2, 0, 2)
            start(kk + 4, 0)
            fin(kk + 3, 1, 3)

            @pl.when(kk + 5 < MNCH)
            def _():
                start(kk + 5, 1)

            return c

        lax.fori_loop(0, (MNCH - 2) // 4, body, 0)
        kk0 = MNCH - 2
        fin(kk0, 0, 0)
        fin(kk0 + 1, 1, 1)
        for q in range(4):
            di, mv, sdi, ss = ring[q]
            pltpu.make_async_copy(mv, agg_s.at[di], ss).wait()
        plsc.subcore_barrier()
        pltpu.sync_copy(agg_s.at[pl.ds(sid * RT, RT)],
                        out_h.at[cid, pl.ds(sid * RT, RT)])

    return k(p, eproj, src, dst, zeros_n)


def _sc_pool(node, batchp, zeros_g):
    @functools.partial(
        pl.kernel,
        out_type=jax.ShapeDtypeStruct((NC, GR, 16), _f32),
        mesh=_mesh(),
        scratch_types=[pltpu.VMEM((CCH,), jnp.int32),
                       pltpu.VMEM((CCH, 16), _f32),
                       pltpu.VMEM_SHARED((GR, 16), _f32),
                       pltpu.SemaphoreType.DMA],
        compiler_params=pltpu.CompilerParams(use_tc_tiling_on_sc=False),
    )
    def k(node_h, b_h, z_h, out_h, i_v, r_v, agg_s, sem):
        cid = lax.axis_index("c")
        sid = lax.axis_index("s")
        w = sid * NC + cid

        @pl.when(sid == 0)
        def _():
            pltpu.sync_copy(z_h, agg_s)

        plsc.subcore_barrier()

        def body(kk, c):
            b = pl.multiple_of(w * PWN + kk * CCH, 8)
            pltpu.sync_copy(b_h.at[pl.ds(b, CCH)], i_v)
            pltpu.sync_copy(node_h.at[pl.ds(b, CCH)], r_v)
            pltpu.sync_copy(r_v, agg_s.at[i_v], add=True)
            return c

        lax.fori_loop(0, NCH_N, body, 0)
        plsc.subcore_barrier()

        @pl.when(sid == 0)
        def _():
            pltpu.sync_copy(agg_s, out_h.at[cid])

    return k(node, batchp, zeros_g)


def _dot(a, b):
    return jnp.dot(a, b, preferred_element_type=_f32)


def _onehot(rows, cols, target_fn):
    r = lax.broadcasted_iota(jnp.int32, (rows, cols), 0)
    c = lax.broadcasted_iota(jnp.int32, (rows, cols), 1)
    return jnp.where((c == target_fn(r)) & (r < 100), 1.0, 0.0).astype(_f32)


def _tc_htab(ce_pad, pt_pad, wp, pe, ge):
    def body(ce_r, pt_r, wp_r, pe_r, ge_r, out_r):
        phys = _dot(pt_r[...], wp_r[...])
        phys = phys * jax.nn.sigmoid(phys)
        per = _dot(_onehot(128, 10, lambda r: jnp.clip(r // 18, 0, 9)),
                   pe_r[...])
        grp = _dot(_onehot(128, 18, lambda r: r % 18), ge_r[...])
        out_r[...] = jnp.concatenate([ce_r[...], phys, per, grp], axis=1)

    return pl.pallas_call(
        body, out_shape=jax.ShapeDtypeStruct((128, 128), _f32),
    )(ce_pad, pt_pad, wp, pe, ge)


def _tc_edge(rel, we_rbf, we_rel, wbot):
    nw = wbot.shape[0]

    def body(rel_r, wa_r, wb_r, wm_r, *outs):
        rel_v = rel_r[...]
        d2 = jnp.sum(rel_v * rel_v, axis=1, keepdims=True)
        dd = jnp.sqrt(d2) + 1e-8
        offs = lax.broadcasted_iota(jnp.int32, (BE, 64), 1).astype(_f32) * DELTA
        rbf = jnp.exp(COEFF * (dd - offs) ** 2)
        unit = rel_v / dd
        el = _dot(rbf, wa_r[...]) + _dot(unit, wb_r[...])
        e = el * jax.nn.sigmoid(el)
        for i, o in enumerate(outs):
            o[...] = _dot(e, wm_r[i]).astype(jnp.bfloat16)

    grid = E // BE
    ob = [jax.ShapeDtypeStruct((E, H), jnp.bfloat16)] * nw
    return pl.pallas_call(
        body,
        grid=(grid,),
        in_specs=[pl.BlockSpec((BE, 16), lambda i: (i, 0)),
                  pl.BlockSpec((64, H), lambda i: (0, 0)),
                  pl.BlockSpec((16, H), lambda i: (0, 0)),
                  pl.BlockSpec((nw, H, H), lambda i: (0, 0, 0))],
        out_specs=[pl.BlockSpec((BE, H), lambda i: (i, 0))] * nw,
        out_shape=ob,
    )(rel, we_rbf, we_rel, wbot)


def _tc_matmul(x, w):
    def body(x_r, w_r, o_r):
        o_r[...] = _dot(x_r[...], w_r[...])

    return pl.pallas_call(
        body,
        grid=(NPAD // BN,),
        in_specs=[pl.BlockSpec((BN, H), lambda i: (i, 0)),
                  pl.BlockSpec((H, H), lambda i: (0, 0))],
        out_specs=pl.BlockSpec((BN, H), lambda i: (i, 0)),
        out_shape=jax.ShapeDtypeStruct((NPAD, H), _f32),
    )(x, w)


def _tc_update(h, a0, a1, wu, wt):
    def body(h_r, a0_r, a1_r, wu_r, wt_r, hn_r, p_r):
        u = _dot(a0_r[...] + a1_r[...], wu_r[...])
        hn = h_r[...] + u * jax.nn.sigmoid(u)
        hn_r[...] = hn
        p_r[...] = _dot(hn, wt_r[...])

    return pl.pallas_call(
        body,
        grid=(NPAD // BN,),
        in_specs=[pl.BlockSpec((BN, H), lambda i: (i, 0))] * 3
        + [pl.BlockSpec((H, H), lambda i: (0, 0))] * 2,
        out_specs=[pl.BlockSpec((BN, H), lambda i: (i, 0))] * 2,
        out_shape=[jax.ShapeDtypeStruct((NPAD, H), _f32)] * 2,
    )(h, a0, a1, wu, wt)


def _tc_final_node(h, a0, a1, wu, wo1, wo2p):
    def body(h_r, a0_r, a1_r, wu_r, wo1_r, wo2_r, node_r):
        u = _dot(a0_r[...] + a1_r[...], wu_r[...])
        hn = h_r[...] + u * jax.nn.sigmoid(u)
        t = _dot(hn, wo1_r[...])
        t = t * jax.nn.sigmoid(t)
        lane = lax.broadcasted_iota(jnp.int32, (BN, 16), 1)
        ones_col = jnp.where(lane == 1, 1.0, 0.0).astype(_f32)
        node_r[...] = _dot(t, wo2_r[...]) + ones_col

    return pl.pallas_call(
        body,
        grid=(NPAD // BN,),
        in_specs=[pl.BlockSpec((BN, H), lambda i: (i, 0))] * 3
        + [pl.BlockSpec((H, H), lambda i: (0, 0)),
           pl.BlockSpec((H, 64), lambda i: (0, 0)),
           pl.BlockSpec((64, 16), lambda i: (0, 0))],
        out_specs=pl.BlockSpec((BN, 16), lambda i: (i, 0)),
        out_shape=jax.ShapeDtypeStruct((NPAD, 16), _f32),
    )(h, a0, a1, wu, wo1, wo2p)


def _tc_finalize(pool):
    def body(p_r, o_r):
        s = jnp.sum(p_r[...], axis=0)
        r16 = lax.broadcasted_iota(jnp.int32, (16, 16), 0)
        sel0 = jnp.where(r16 == 0, 1.0, 0.0).astype(_f32)
        sel1 = jnp.where(r16 == 1, 1.0, 0.0).astype(_f32)
        o_r[...] = _dot(s, sel0) / jnp.maximum(_dot(s, sel1), 1.0)

    return pl.pallas_call(
        body, out_shape=jax.ShapeDtypeStruct((GR, 16), _f32),
    )(pool)


def kernel(z, pos, edge_index, batch, comp_emb, period_emb, group_emb,
           phys_table, W_phys, W_edge, W_msg, W_upd, W_out1, W_out2):
    srcf = edge_index[0].astype(jnp.int32)
    dstf = edge_index[1].astype(jnp.int32)
    src3 = srcf.reshape(NW, NCH_E, CCH)
    dst3 = dstf.reshape(NW, NCH_E, CCH)
    z_pad = jnp.concatenate([z.astype(jnp.int32),
                             jnp.zeros((NPAD - N,), jnp.int32)])
    batch_pad = jnp.concatenate([batch.astype(jnp.int32),
                                 jnp.full((NPAD - N,), NGRAPH, jnp.int32)])
    posp = jnp.pad(pos.astype(_f32), ((0, 0), (0, 13)))
    ce_pad = jnp.pad(comp_emb, ((0, 28), (0, 0)))
    pt_pad = jnp.pad(phys_table, ((0, 28), (0, 0)))
    we_rbf = jnp.pad(W_edge[:NG], ((0, 64 - NG), (0, 0)))
    we_rel = jnp.pad(W_edge[NG:NG + 3], ((0, 13), (0, 0)))
    wtop = W_msg[:, :H, :]
    wbot = W_msg[:, H:, :]
    wo2p = jnp.pad(W_out2, ((0, 0), (0, 15)))
    zeros_n = jnp.zeros((NPAD, H), _f32)
    zeros_g = jnp.zeros((GR, 16), _f32)

    htab = _tc_htab(ce_pad, pt_pad, W_phys, period_emb, group_emb)
    h = _sc_gather_rows(htab, z_pad, H, NCH_N, PWN)
    rel = _sc_rel(posp, src3, dst3)
    perm = np.concatenate(
        [32 * j + (np.arange(32) // 2) + 16 * (np.arange(32) % 2)
         for j in range(4)])
    wbot_p = wbot[:, :, perm]
    ep16 = _tc_edge(rel, we_rbf, we_rel, wbot_p)
    ep = [lax.bitcast_convert_type(e.reshape(E, H // 2, 2), _f32)
          for e in ep16]
    p = _tc_matmul(h, wtop[0])

    for i in range(4):
        aggp = _sc_msg(p, ep[i], srcf, dstf, zeros_n)
        if i < 3:
            h, p = _tc_update(h, aggp[0], aggp[1], W_upd[i], wtop[i + 1])
        else:
            node = _tc_final_node(h, aggp[0], aggp[1], W_upd[3], W_out1, wo2p)

    pool = _sc_pool(node, batch_pad, zeros_g)
    res = _tc_finalize(pool)
    return res[:NGRAPH, 0:1]


# speedup vs baseline: 2.9270x; 2.9270x over previous
import functools

import numpy as np
import jax
import jax.numpy as jnp
from jax import lax
from jax.experimental import pallas as pl
from jax.experimental.pallas import tpu as pltpu
from jax.experimental.pallas import tpu_sc as plsc

N = 10000
E = 320000
H = 128
NG = 50
CUTOFF = 6.0
NGRAPH = 128
GR = 136
NPAD = 10240
NC, NS = 2, 16
NW = NC * NS
EW = E // NW
CCH = 80
NCH_E = EW // CCH
PWN = NPAD // NW
NCH_N = PWN // CCH
RT = NPAD // NS
DELTA = CUTOFF / (NG - 1)
COEFF = -0.5 / DELTA**2
BE = 1280
BN = 1024

_f32 = jnp.float32


def _mesh():
    return plsc.VectorSubcoreMesh(core_axis_name="c", subcore_axis_name="s",
                                  num_cores=NC, num_subcores=NS)


def _wid():
    return lax.axis_index("s") * NC + lax.axis_index("c")


def _sc_gather_rows(table, idx, d, nch, per_w):
    ntot = idx.shape[0]

    @functools.partial(
        pl.kernel,
        out_type=jax.ShapeDtypeStruct((ntot, d), _f32),
        mesh=_mesh(),
        scratch_types=[pltpu.VMEM((CCH,), jnp.int32),
                       pltpu.VMEM((CCH, d), _f32),
                       pltpu.SemaphoreType.DMA],
    )
    def k(tab_h, idx_h, out_h, idx_v, row_v, sem):
        base0 = _wid() * per_w

        def body(kk, c):
            b = pl.multiple_of(base0 + kk * CCH, 8)
            pltpu.sync_copy(idx_h.at[pl.ds(b, CCH)], idx_v)
            pltpu.async_copy(tab_h.at[idx_v], row_v, sem).wait()
            pltpu.sync_copy(row_v, out_h.at[pl.ds(b, CCH)])
            return c

        lax.fori_loop(0, nch, body, 0)

    return k(table, idx)


def _sc_rel(posp, src3, dst3):
    @functools.partial(
        pl.kernel,
        out_type=jax.ShapeDtypeStruct((E, 16), _f32),
        mesh=_mesh(),
        scratch_types=[pltpu.VMEM((NCH_E, CCH), jnp.int32),
                       pltpu.VMEM((NCH_E, CCH), jnp.int32),
                       pltpu.VMEM((CCH, 16), _f32),
                       pltpu.VMEM((CCH, 16), _f32),
                       pltpu.VMEM((CCH, 16), _f32),
                       pltpu.VMEM((CCH, 16), _f32),
                       pltpu.SemaphoreType.DMA,
                       pltpu.SemaphoreType.DMA,
                       pltpu.SemaphoreType.DMA,
                       pltpu.SemaphoreType.DMA],
        compiler_params=pltpu.CompilerParams(use_tc_tiling_on_sc=False),
    )
    def k(pos_h, src_h, dst_h, out_h, si_v, di_v, a0, b0, a1, b1,
          sg0, sl0, sg1, sl1):
        w = _wid()
        base0 = w * EW
        bufs = ((a0, b0, sg0, sl0), (a1, b1, sg1, sl1))
        pltpu.sync_copy(src_h.at[w], si_v)
        pltpu.sync_copy(dst_h.at[w], di_v)

        def start(kk, b):
            av, bv, sg, sl = bufs[b]
            pltpu.async_copy(pos_h.at[si_v.at[kk]], av, sg)
            pltpu.async_copy(pos_h.at[di_v.at[kk]], bv, sl)

        def fin(kk, b):
            av, bv, sg, sl = bufs[b]
            pltpu.make_async_copy(pos_h.at[si_v.at[kk]], av, sg).wait()
            pltpu.make_async_copy(pos_h.at[di_v.at[kk]], bv, sl).wait()

            def row(r, cc):
                bv[r, :] = bv[r, :] - av[r, :]
                return cc

            lax.fori_loop(0, CCH, row, 0)
            off = pl.multiple_of(base0 + kk * CCH, 8)
            pltpu.sync_copy(bv, out_h.at[pl.ds(off, CCH)])

        start(0, 0)
        start(1, 1)

        def body(j, c):
            kk = j * 2
            fin(kk, 0)
            start(kk + 2, 0)
            fin(kk + 1, 1)

            @pl.when(kk + 3 < NCH_E)
            def _():
                start(kk + 3, 1)

            return c

        lax.fori_loop(0, NCH_E // 2, body, 0)
        fin(NCH_E - 1, 0)

    return k(posp, src3, dst3)


MC = 40
MNCH = EW // MC


def _sc_msg(p, eproj, src, dst, zeros_n):
    @functools.partial(
        pl.kernel,
        out_type=jax.ShapeDtypeStruct((NC, NPAD, H), _f32),
        mesh=_mesh(),
        scratch_types=[pltpu.VMEM((MC,), jnp.int32),
                       pltpu.VMEM((MC,), jnp.int32),
                       pltpu.VMEM((MC, H), _f32),
                       pltpu.VMEM((MC, H), _f32),
                       pltpu.VMEM((MC, H), _f32),
                       pltpu.VMEM((MC, H), _f32),
                       pltpu.VMEM((MC,), jnp.int32),
                       pltpu.VMEM((MC,), jnp.int32),
                       pltpu.VMEM((MC,), jnp.int32),
                       pltpu.VMEM((MC,), jnp.int32),
                       pltpu.VMEM((MC, H), _f32),
                       pltpu.VMEM((MC, H), _f32),
                       pltpu.VMEM((MC, H), _f32),
                       pltpu.VMEM((MC, H), _f32),
                       pltpu.VMEM_SHARED((NPAD, H), _f32),
                       pltpu.SemaphoreType.DMA,
                       pltpu.SemaphoreType.DMA,
                       pltpu.SemaphoreType.DMA,
                       pltpu.SemaphoreType.DMA,
                       pltpu.SemaphoreType.DMA,
                       pltpu.SemaphoreType.DMA,
                       pltpu.SemaphoreType.DMA,
                       pltpu.SemaphoreType.DMA,
                       pltpu.SemaphoreType.DMA,
                       pltpu.SemaphoreType.DMA,
                       pltpu.SemaphoreType.DMA,
                       pltpu.SemaphoreType.DMA,
                       pltpu.SemaphoreType.DMA,
                       pltpu.SemaphoreType.DMA],
    )
    def k(p_h, ep_h, src_h, dst_h, z_h, out_h,
          si0, si1, a0, b0, a1, b1,
          di0, di1, di2, di3, m0, m1, m2, m3, agg_s,
          ssi0, ssi1, sg0, sl0, sg1, sl1,
          sdi0, sdi1, sdi2, sdi3, ss0, ss1, ss2, ss3):
        cid = lax.axis_index("c")
        sid = lax.axis_index("s")
        w = sid * NC + cid
        base0 = w * MC * MNCH
        pltpu.sync_copy(z_h.at[pl.ds(sid * RT, RT)],
                        agg_s.at[pl.ds(sid * RT, RT)])
        data = ((si0, a0, b0, ssi0, sg0, sl0),
                (si1, a1, b1, ssi1, sg1, sl1))
        ring = ((di0, m0, sdi0, ss0), (di1, m1, sdi1, ss1),
                (di2, m2, sdi2, ss2), (di3, m3, sdi3, ss3))

        def sslice(h_ref, kk):
            return h_ref.at[pl.ds(pl.multiple_of(base0 + kk * MC, 8), MC)]

        def start(kk, d):
            si, av, bv, ssi, sg, sl = data[d]
            pltpu.make_async_copy(sslice(src_h, kk), si, ssi).wait()
            pltpu.async_copy(p_h.at[si], av, sg)
            pltpu.async_copy(sslice(ep_h, kk), bv, sl)

        def fin(kk, d, q):
            si, av, bv, ssi, sg, sl = data[d]
            di, mv, sdi, ss = ring[q]
            pltpu.make_async_copy(p_h.at[si], av, sg).wait()
            pltpu.make_async_copy(sslice(ep_h, kk), bv, sl).wait()

            @pl.when(kk + 2 < MNCH)
            def _():
                pltpu.async_copy(sslice(src_h, kk + 2), si, ssi)

            @pl.when(kk >= 4)
            def _():
                pltpu.make_async_copy(mv, agg_s.at[di], ss).wait()

            pltpu.async_copy(sslice(dst_h, kk), di, sdi)

            def row(r, cc):
                for j in range(H // 16):
                    v = av[r, pl.ds(j * 16, 16)] + bv[r, pl.ds(j * 16, 16)]
                    mv[r, pl.ds(j * 16, 16)] = v / (1.0 + jnp.exp(-v))
                return cc

            lax.fori_loop(0, MC, row, 0)
            pltpu.make_async_copy(sslice(dst_h, kk), di, sdi).wait()
            pltpu.async_copy(mv, agg_s.at[di], ss, add=True)

        pltpu.async_copy(sslice(src_h, 0), si0, ssi0)
        pltpu.async_copy(sslice(src_h, 1), si1, ssi1)
        plsc.subcore_barrier()
        start(0, 0)
        start(1, 1)

        def body(j, c):
            kk = j * 4
            fin(kk, 0, 0)
            start(kk + 2, 0)
            fin(kk + 1, 1, 1)
            start(kk + 3, 1)
            fin(kk + 2, 0, 2)
            start(kk + 4, 0)
            fin(kk + 3, 1, 3)

            @pl.when(kk + 5 < MNCH)
            def _():
                start(kk + 5, 1)

            return c

        lax.fori_loop(0, (MNCH - 2) // 4, body, 0)
        kk0 = MNCH - 2
        fin(kk0, 0, 0)
        fin(kk0 + 1, 1, 1)
        for q in range(4):
            di, mv, sdi, ss = ring[q]
            pltpu.make_async_copy(mv, agg_s.at[di], ss).wait()
        plsc.subcore_barrier()
        pltpu.sync_copy(agg_s.at[pl.ds(sid * RT, RT)],
                        out_h.at[cid, pl.ds(sid * RT, RT)])

    return k(p, eproj, src, dst, zeros_n)


def _sc_pool(node, batchp, zeros_g):
    @functools.partial(
        pl.kernel,
        out_type=jax.ShapeDtypeStruct((NC, GR, 16), _f32),
        mesh=_mesh(),
        scratch_types=[pltpu.VMEM((CCH,), jnp.int32),
                       pltpu.VMEM((CCH, 16), _f32),
                       pltpu.VMEM_SHARED((GR, 16), _f32),
                       pltpu.SemaphoreType.DMA],
        compiler_params=pltpu.CompilerParams(use_tc_tiling_on_sc=False),
    )
    def k(node_h, b_h, z_h, out_h, i_v, r_v, agg_s, sem):
        cid = lax.axis_index("c")
        sid = lax.axis_index("s")
        w = sid * NC + cid

        @pl.when(sid == 0)
        def _():
            pltpu.sync_copy(z_h, agg_s)

        plsc.subcore_barrier()

        def body(kk, c):
            b = pl.multiple_of(w * PWN + kk * CCH, 8)
            pltpu.sync_copy(b_h.at[pl.ds(b, CCH)], i_v)
            pltpu.sync_copy(node_h.at[pl.ds(b, CCH)], r_v)
            pltpu.sync_copy(r_v, agg_s.at[i_v], add=True)
            return c

        lax.fori_loop(0, NCH_N, body, 0)
        plsc.subcore_barrier()

        @pl.when(sid == 0)
        def _():
            pltpu.sync_copy(agg_s, out_h.at[cid])

    return k(node, batchp, zeros_g)


def _dot(a, b):
    return jnp.dot(a, b, preferred_element_type=_f32)


def _onehot(rows, cols, target_fn):
    r = lax.broadcasted_iota(jnp.int32, (rows, cols), 0)
    c = lax.broadcasted_iota(jnp.int32, (rows, cols), 1)
    return jnp.where((c == target_fn(r)) & (r < 100), 1.0, 0.0).astype(_f32)


def _tc_htab(ce_pad, pt_pad, wp, pe, ge):
    def body(ce_r, pt_r, wp_r, pe_r, ge_r, out_r):
        phys = _dot(pt_r[...], wp_r[...])
        phys = phys * jax.nn.sigmoid(phys)
        per = _dot(_onehot(128, 10, lambda r: jnp.clip(r // 18, 0, 9)),
                   pe_r[...])
        grp = _dot(_onehot(128, 18, lambda r: r % 18), ge_r[...])
        out_r[...] = jnp.concatenate([ce_r[...], phys, per, grp], axis=1)

    return pl.pallas_call(
        body, out_shape=jax.ShapeDtypeStruct((128, 128), _f32),
    )(ce_pad, pt_pad, wp, pe, ge)


def _tc_edge(rel, we_rbf, we_rel, wbot):
    nw = wbot.shape[0]

    def body(rel_r, wa_r, wb_r, wm_r, *outs):
        rel_v = rel_r[...]
        d2 = jnp.sum(rel_v * rel_v, axis=1, keepdims=True)
        dd = jnp.sqrt(d2) + 1e-8
        offs = lax.broadcasted_iota(jnp.int32, (BE, 64), 1).astype(_f32) * DELTA
        rbf = jnp.exp(COEFF * (dd - offs) ** 2)
        unit = rel_v / dd
        el = _dot(rbf, wa_r[...]) + _dot(unit, wb_r[...])
        e = el * jax.nn.sigmoid(el)
        for i, o in enumerate(outs):
            o[...] = _dot(e, wm_r[i])

    grid = E // BE
    ob = [jax.ShapeDtypeStruct((E, H), _f32)] * nw
    return pl.pallas_call(
        body,
        grid=(grid,),
        in_specs=[pl.BlockSpec((BE, 16), lambda i: (i, 0)),
                  pl.BlockSpec((64, H), lambda i: (0, 0)),
                  pl.BlockSpec((16, H), lambda i: (0, 0)),
                  pl.BlockSpec((nw, H, H), lambda i: (0, 0, 0))],
        out_specs=[pl.BlockSpec((BE, H), lambda i: (i, 0))] * nw,
        out_shape=ob,
    )(rel, we_rbf, we_rel, wbot)


def _tc_matmul(x, w):
    def body(x_r, w_r, o_r):
        o_r[...] = _dot(x_r[...], w_r[...])

    return pl.pallas_call(
        body,
        grid=(NPAD // BN,),
        in_specs=[pl.BlockSpec((BN, H), lambda i: (i, 0)),
                  pl.BlockSpec((H, H), lambda i: (0, 0))],
        out_specs=pl.BlockSpec((BN, H), lambda i: (i, 0)),
        out_shape=jax.ShapeDtypeStruct((NPAD, H), _f32),
    )(x, w)


def _tc_update(h, a0, a1, wu, wt):
    def body(h_r, a0_r, a1_r, wu_r, wt_r, hn_r, p_r):
        u = _dot(a0_r[...] + a1_r[...], wu_r[...])
        hn = h_r[...] + u * jax.nn.sigmoid(u)
        hn_r[...] = hn
        p_r[...] = _dot(hn, wt_r[...])

    return pl.pallas_call(
        body,
        grid=(NPAD // BN,),
        in_specs=[pl.BlockSpec((BN, H), lambda i: (i, 0))] * 3
        + [pl.BlockSpec((H, H), lambda i: (0, 0))] * 2,
        out_specs=[pl.BlockSpec((BN, H), lambda i: (i, 0))] * 2,
        out_shape=[jax.ShapeDtypeStruct((NPAD, H), _f32)] * 2,
    )(h, a0, a1, wu, wt)


def _tc_final_node(h, a0, a1, wu, wo1, wo2p):
    def body(h_r, a0_r, a1_r, wu_r, wo1_r, wo2_r, node_r):
        u = _dot(a0_r[...] + a1_r[...], wu_r[...])
        hn = h_r[...] + u * jax.nn.sigmoid(u)
        t = _dot(hn, wo1_r[...])
        t = t * jax.nn.sigmoid(t)
        lane = lax.broadcasted_iota(jnp.int32, (BN, 16), 1)
        ones_col = jnp.where(lane == 1, 1.0, 0.0).astype(_f32)
        node_r[...] = _dot(t, wo2_r[...]) + ones_col

    return pl.pallas_call(
        body,
        grid=(NPAD // BN,),
        in_specs=[pl.BlockSpec((BN, H), lambda i: (i, 0))] * 3
        + [pl.BlockSpec((H, H), lambda i: (0, 0)),
           pl.BlockSpec((H, 64), lambda i: (0, 0)),
           pl.BlockSpec((64, 16), lambda i: (0, 0))],
        out_specs=pl.BlockSpec((BN, 16), lambda i: (i, 0)),
        out_shape=jax.ShapeDtypeStruct((NPAD, 16), _f32),
    )(h, a0, a1, wu, wo1, wo2p)


def _tc_finalize(pool):
    def body(p_r, o_r):
        s = jnp.sum(p_r[...], axis=0)
        r16 = lax.broadcasted_iota(jnp.int32, (16, 16), 0)
        sel0 = jnp.where(r16 == 0, 1.0, 0.0).astype(_f32)
        sel1 = jnp.where(r16 == 1, 1.0, 0.0).astype(_f32)
        o_r[...] = _dot(s, sel0) / jnp.maximum(_dot(s, sel1), 1.0)

    return pl.pallas_call(
        body, out_shape=jax.ShapeDtypeStruct((GR, 16), _f32),
    )(pool)


def kernel(z, pos, edge_index, batch, comp_emb, period_emb, group_emb,
           phys_table, W_phys, W_edge, W_msg, W_upd, W_out1, W_out2):
    srcf = edge_index[0].astype(jnp.int32)
    dstf = edge_index[1].astype(jnp.int32)
    src3 = srcf.reshape(NW, NCH_E, CCH)
    dst3 = dstf.reshape(NW, NCH_E, CCH)
    z_pad = jnp.concatenate([z.astype(jnp.int32),
                             jnp.zeros((NPAD - N,), jnp.int32)])
    batch_pad = jnp.concatenate([batch.astype(jnp.int32),
                                 jnp.full((NPAD - N,), NGRAPH, jnp.int32)])
    posp = jnp.pad(pos.astype(_f32), ((0, 0), (0, 13)))
    ce_pad = jnp.pad(comp_emb, ((0, 28), (0, 0)))
    pt_pad = jnp.pad(phys_table, ((0, 28), (0, 0)))
    we_rbf = jnp.pad(W_edge[:NG], ((0, 64 - NG), (0, 0)))
    we_rel = jnp.pad(W_edge[NG:NG + 3], ((0, 13), (0, 0)))
    wtop = W_msg[:, :H, :]
    wbot = W_msg[:, H:, :]
    wo2p = jnp.pad(W_out2, ((0, 0), (0, 15)))
    zeros_n = jnp.zeros((NPAD, H), _f32)
    zeros_g = jnp.zeros((GR, 16), _f32)

    htab = _tc_htab(ce_pad, pt_pad, W_phys, period_emb, group_emb)
    h = _sc_gather_rows(htab, z_pad, H, NCH_N, PWN)
    rel = _sc_rel(posp, src3, dst3)
    ep = _tc_edge(rel, we_rbf, we_rel, wbot)
    p = _tc_matmul(h, wtop[0])

    for i in range(4):
        aggp = _sc_msg(p, ep[i], srcf, dstf, zeros_n)
        if i < 3:
            h, p = _tc_update(h, aggp[0], aggp[1], W_upd[i], wtop[i + 1])
        else:
            node = _tc_final_node(h, aggp[0], aggp[1], W_upd[3], W_out1, wo2p)

    pool = _sc_pool(node, batch_pad, zeros_g)
    res = _tc_finalize(pool)
    return res[:NGRAPH, 0:1]


# speedup vs baseline: 2.9329x; 1.0020x over previous
import functools

import numpy as np
import jax
import jax.numpy as jnp
from jax import lax
from jax.experimental import pallas as pl
from jax.experimental.pallas import tpu as pltpu
from jax.experimental.pallas import tpu_sc as plsc

N = 10000
E = 320000
H = 128
NG = 50
CUTOFF = 6.0
NGRAPH = 128
GR = 136
NPAD = 10240
NC, NS = 2, 16
NW = NC * NS
EW = E // NW
CCH = 80
NCH_E = EW // CCH
PWN = NPAD // NW
NCH_N = PWN // CCH
RT = NPAD // NS
DELTA = CUTOFF / (NG - 1)
COEFF = -0.5 / DELTA**2
BE = 1280
BN = 1024

_f32 = jnp.float32


def _mesh():
    return plsc.VectorSubcoreMesh(core_axis_name="c", subcore_axis_name="s",
                                  num_cores=NC, num_subcores=NS)


def _wid():
    return lax.axis_index("s") * NC + lax.axis_index("c")


def _sc_gather2(taba, tabb, idx):
    @functools.partial(
        pl.kernel,
        out_type=(jax.ShapeDtypeStruct((NPAD, H), _f32),
                  jax.ShapeDtypeStruct((NPAD, H), _f32)),
        mesh=_mesh(),
        scratch_types=[pltpu.VMEM((CCH,), jnp.int32),
                       pltpu.VMEM((CCH, H), _f32),
                       pltpu.VMEM((CCH, H), _f32),
                       pltpu.SemaphoreType.DMA,
                       pltpu.SemaphoreType.DMA],
    )
    def k(taba_h, tabb_h, idx_h, outa_h, outb_h, idx_v, ra_v, rb_v, s1, s2):
        base0 = _wid() * PWN

        def body(kk, c):
            b = pl.multiple_of(base0 + kk * CCH, 8)
            pltpu.sync_copy(idx_h.at[pl.ds(b, CCH)], idx_v)
            cpa = pltpu.async_copy(taba_h.at[idx_v], ra_v, s1)
            cpb = pltpu.async_copy(tabb_h.at[idx_v], rb_v, s2)
            cpa.wait()
            cpb.wait()
            pltpu.sync_copy(ra_v, outa_h.at[pl.ds(b, CCH)])
            pltpu.sync_copy(rb_v, outb_h.at[pl.ds(b, CCH)])
            return c

        lax.fori_loop(0, NCH_N, body, 0)

    return k(taba, tabb, idx)


def _sc_rel(posp, src3, dst3):
    @functools.partial(
        pl.kernel,
        out_type=jax.ShapeDtypeStruct((E, 16), _f32),
        mesh=_mesh(),
        scratch_types=[pltpu.VMEM((NCH_E, CCH), jnp.int32),
                       pltpu.VMEM((NCH_E, CCH), jnp.int32),
                       pltpu.VMEM((CCH, 16), _f32),
                       pltpu.VMEM((CCH, 16), _f32),
                       pltpu.VMEM((CCH, 16), _f32),
                       pltpu.VMEM((CCH, 16), _f32),
                       pltpu.SemaphoreType.DMA,
                       pltpu.SemaphoreType.DMA,
                       pltpu.SemaphoreType.DMA,
                       pltpu.SemaphoreType.DMA],
        compiler_params=pltpu.CompilerParams(use_tc_tiling_on_sc=False),
    )
    def k(pos_h, src_h, dst_h, out_h, si_v, di_v, a0, b0, a1, b1,
          sg0, sl0, sg1, sl1):
        w = _wid()
        base0 = w * EW
        bufs = ((a0, b0, sg0, sl0), (a1, b1, sg1, sl1))
        pltpu.sync_copy(src_h.at[w], si_v)
        pltpu.sync_copy(dst_h.at[w], di_v)

        def start(kk, b):
            av, bv, sg, sl = bufs[b]
            pltpu.async_copy(pos_h.at[si_v.at[kk]], av, sg)
            pltpu.async_copy(pos_h.at[di_v.at[kk]], bv, sl)

        def fin(kk, b):
            av, bv, sg, sl = bufs[b]
            pltpu.make_async_copy(pos_h.at[si_v.at[kk]], av, sg).wait()
            pltpu.make_async_copy(pos_h.at[di_v.at[kk]], bv, sl).wait()

            def row(r, cc):
                bv[r, :] = bv[r, :] - av[r, :]
                return cc

            lax.fori_loop(0, CCH, row, 0)
            off = pl.multiple_of(base0 + kk * CCH, 8)
            pltpu.sync_copy(bv, out_h.at[pl.ds(off, CCH)])

        start(0, 0)
        start(1, 1)

        def body(j, c):
            kk = j * 2
            fin(kk, 0)
            start(kk + 2, 0)
            fin(kk + 1, 1)

            @pl.when(kk + 3 < NCH_E)
            def _():
                start(kk + 3, 1)

            return c

        lax.fori_loop(0, NCH_E // 2, body, 0)
        fin(NCH_E - 1, 0)

    return k(posp, src3, dst3)


MC = 40
MNCH = EW // MC


def _sc_msg(p, eproj, src, dst, zeros_n):
    @functools.partial(
        pl.kernel,
        out_type=jax.ShapeDtypeStruct((NC, NPAD, H), _f32),
        mesh=_mesh(),
        scratch_types=[pltpu.VMEM((MC,), jnp.int32),
                       pltpu.VMEM((MC,), jnp.int32),
                       pltpu.VMEM((MC, H), _f32),
                       pltpu.VMEM((MC, H), _f32),
                       pltpu.VMEM((MC, H), _f32),
                       pltpu.VMEM((MC, H), _f32),
                       pltpu.VMEM((MC,), jnp.int32),
                       pltpu.VMEM((MC,), jnp.int32),
                       pltpu.VMEM((MC,), jnp.int32),
                       pltpu.VMEM((MC,), jnp.int32),
                       pltpu.VMEM((MC, H), _f32),
                       pltpu.VMEM((MC, H), _f32),
                       pltpu.VMEM((MC, H), _f32),
                       pltpu.VMEM((MC, H), _f32),
                       pltpu.VMEM_SHARED((NPAD, H), _f32),
                       pltpu.SemaphoreType.DMA,
                       pltpu.SemaphoreType.DMA,
                       pltpu.SemaphoreType.DMA,
                       pltpu.SemaphoreType.DMA,
                       pltpu.SemaphoreType.DMA,
                       pltpu.SemaphoreType.DMA,
                       pltpu.SemaphoreType.DMA,
                       pltpu.SemaphoreType.DMA,
                       pltpu.SemaphoreType.DMA,
                       pltpu.SemaphoreType.DMA,
                       pltpu.SemaphoreType.DMA,
                       pltpu.SemaphoreType.DMA,
                       pltpu.SemaphoreType.DMA,
                       pltpu.SemaphoreType.DMA],
    )
    def k(p_h, ep_h, src_h, dst_h, z_h, out_h,
          si0, si1, a0, b0, a1, b1,
          di0, di1, di2, di3, m0, m1, m2, m3, agg_s,
          ssi0, ssi1, sg0, sl0, sg1, sl1,
          sdi0, sdi1, sdi2, sdi3, ss0, ss1, ss2, ss3):
        cid = lax.axis_index("c")
        sid = lax.axis_index("s")
        w = sid * NC + cid
        base0 = w * MC * MNCH
        pltpu.sync_copy(z_h.at[pl.ds(sid * RT, RT)],
                        agg_s.at[pl.ds(sid * RT, RT)])
        data = ((si0, a0, b0, ssi0, sg0, sl0),
                (si1, a1, b1, ssi1, sg1, sl1))
        ring = ((di0, m0, sdi0, ss0), (di1, m1, sdi1, ss1),
                (di2, m2, sdi2, ss2), (di3, m3, sdi3, ss3))

        def sslice(h_ref, kk):
            return h_ref.at[pl.ds(pl.multiple_of(base0 + kk * MC, 8), MC)]

        def start(kk, d):
            si, av, bv, ssi, sg, sl = data[d]
            pltpu.make_async_copy(sslice(src_h, kk), si, ssi).wait()
            pltpu.async_copy(p_h.at[si], av, sg)
            pltpu.async_copy(sslice(ep_h, kk), bv, sl)

        def fin(kk, d, q):
            si, av, bv, ssi, sg, sl = data[d]
            di, mv, sdi, ss = ring[q]
            pltpu.make_async_copy(p_h.at[si], av, sg).wait()
            pltpu.make_async_copy(sslice(ep_h, kk), bv, sl).wait()

            @pl.when(kk + 2 < MNCH)
            def _():
                pltpu.async_copy(sslice(src_h, kk + 2), si, ssi)

            @pl.when(kk >= 4)
            def _():
                pltpu.make_async_copy(mv, agg_s.at[di], ss).wait()

            pltpu.async_copy(sslice(dst_h, kk), di, sdi)

            def row(r, cc):
                for j in range(H // 16):
                    v = av[r, pl.ds(j * 16, 16)] + bv[r, pl.ds(j * 16, 16)]
                    mv[r, pl.ds(j * 16, 16)] = v / (1.0 + jnp.exp(-v))
                return cc

            lax.fori_loop(0, MC, row, 0)
            pltpu.make_async_copy(sslice(dst_h, kk), di, sdi).wait()
            pltpu.async_copy(mv, agg_s.at[di], ss, add=True)

        pltpu.async_copy(sslice(src_h, 0), si0, ssi0)
        pltpu.async_copy(sslice(src_h, 1), si1, ssi1)
        plsc.subcore_barrier()
        start(0, 0)
        start(1, 1)

        def body(j, c):
            kk = j * 4
            fin(kk, 0, 0)
            start(kk + 2, 0)
            fin(kk + 1, 1, 1)
            start(kk + 3, 1)
            fin(kk + 2, 0, 2)
            start(kk + 4, 0)
            fin(kk + 3, 1, 3)

            @pl.when(kk + 5 < MNCH)
            def _():
                start(kk + 5, 1)

            return c

        lax.fori_loop(0, (MNCH - 2) // 4, body, 0)
        kk0 = MNCH - 2
        fin(kk0, 0, 0)
        fin(kk0 + 1, 1, 1)
        for q in range(4):
            di, mv, sdi, ss = ring[q]
            pltpu.make_async_copy(mv, agg_s.at[di], ss).wait()
        plsc.subcore_barrier()
        pltpu.sync_copy(agg_s.at[pl.ds(sid * RT, RT)],
                        out_h.at[cid, pl.ds(sid * RT, RT)])

    return k(p, eproj, src, dst, zeros_n)


def _sc_pool(node, batchp, zeros_g):
    @functools.partial(
        pl.kernel,
        out_type=jax.ShapeDtypeStruct((NC, GR, 16), _f32),
        mesh=_mesh(),
        scratch_types=[pltpu.VMEM((CCH,), jnp.int32),
                       pltpu.VMEM((CCH, 16), _f32),
                       pltpu.VMEM_SHARED((GR, 16), _f32),
                       pltpu.SemaphoreType.DMA],
        compiler_params=pltpu.CompilerParams(use_tc_tiling_on_sc=False),
    )
    def k(node_h, b_h, z_h, out_h, i_v, r_v, agg_s, sem):
        cid = lax.axis_index("c")
        sid = lax.axis_index("s")
        w = sid * NC + cid

        @pl.when(sid == 0)
        def _():
            pltpu.sync_copy(z_h, agg_s)

        plsc.subcore_barrier()

        def body(kk, c):
            b = pl.multiple_of(w * PWN + kk * CCH, 8)
            pltpu.sync_copy(b_h.at[pl.ds(b, CCH)], i_v)
            pltpu.sync_copy(node_h.at[pl.ds(b, CCH)], r_v)
            pltpu.sync_copy(r_v, agg_s.at[i_v], add=True)
            return c

        lax.fori_loop(0, NCH_N, body, 0)
        plsc.subcore_barrier()

        @pl.when(sid == 0)
        def _():
            pltpu.sync_copy(agg_s, out_h.at[cid])

    return k(node, batchp, zeros_g)


def _dot(a, b):
    return jnp.dot(a, b, preferred_element_type=_f32)


def _onehot(rows, cols, target_fn):
    r = lax.broadcasted_iota(jnp.int32, (rows, cols), 0)
    c = lax.broadcasted_iota(jnp.int32, (rows, cols), 1)
    return jnp.where((c == target_fn(r)) & (r < 100), 1.0, 0.0).astype(_f32)


def _tc_htab(ce_pad, pt_pad, wp, pe, ge, wtop0):
    def body(ce_r, pt_r, wp_r, pe_r, ge_r, wt_r, out_r, outp_r):
        phys = _dot(pt_r[...], wp_r[...])
        phys = phys * jax.nn.sigmoid(phys)
        per = _dot(_onehot(128, 10, lambda r: jnp.clip(r // 18, 0, 9)),
                   pe_r[...])
        grp = _dot(_onehot(128, 18, lambda r: r % 18), ge_r[...])
        tab = jnp.concatenate([ce_r[...], phys, per, grp], axis=1)
        out_r[...] = tab
        outp_r[...] = _dot(tab, wt_r[...])

    return pl.pallas_call(
        body, out_shape=(jax.ShapeDtypeStruct((128, 128), _f32),
                         jax.ShapeDtypeStruct((128, 128), _f32)),
    )(ce_pad, pt_pad, wp, pe, ge, wtop0)


def _tc_edge(rel, we_rbf, we_rel, wbot):
    nw = wbot.shape[0]

    def body(rel_r, wa_r, wb_r, wm_r, *outs):
        rel_v = rel_r[...]
        d2 = jnp.sum(rel_v * rel_v, axis=1, keepdims=True)
        dd = jnp.sqrt(d2) + 1e-8
        offs = lax.broadcasted_iota(jnp.int32, (BE, 64), 1).astype(_f32) * DELTA
        rbf = jnp.exp(COEFF * (dd - offs) ** 2)
        unit = rel_v / dd
        el = _dot(rbf, wa_r[...]) + _dot(unit, wb_r[...])
        e = el * jax.nn.sigmoid(el)
        for i, o in enumerate(outs):
            o[...] = _dot(e, wm_r[i])

    grid = E // BE
    ob = [jax.ShapeDtypeStruct((E, H), _f32)] * nw
    return pl.pallas_call(
        body,
        grid=(grid,),
        in_specs=[pl.BlockSpec((BE, 16), lambda i: (i, 0)),
                  pl.BlockSpec((64, H), lambda i: (0, 0)),
                  pl.BlockSpec((16, H), lambda i: (0, 0)),
                  pl.BlockSpec((nw, H, H), lambda i: (0, 0, 0))],
        out_specs=[pl.BlockSpec((BE, H), lambda i: (i, 0))] * nw,
        out_shape=ob,
    )(rel, we_rbf, we_rel, wbot)


def _tc_update(h, a0, a1, wu, wt):
    def body(h_r, a0_r, a1_r, wu_r, wt_r, hn_r, p_r):
        u = _dot(a0_r[...] + a1_r[...], wu_r[...])
        hn = h_r[...] + u * jax.nn.sigmoid(u)
        hn_r[...] = hn
        p_r[...] = _dot(hn, wt_r[...])

    return pl.pallas_call(
        body,
        grid=(NPAD // BN,),
        in_specs=[pl.BlockSpec((BN, H), lambda i: (i, 0))] * 3
        + [pl.BlockSpec((H, H), lambda i: (0, 0))] * 2,
        out_specs=[pl.BlockSpec((BN, H), lambda i: (i, 0))] * 2,
        out_shape=[jax.ShapeDtypeStruct((NPAD, H), _f32)] * 2,
    )(h, a0, a1, wu, wt)


def _tc_final_node(h, a0, a1, wu, wo1, wo2p):
    def body(h_r, a0_r, a1_r, wu_r, wo1_r, wo2_r, node_r):
        u = _dot(a0_r[...] + a1_r[...], wu_r[...])
        hn = h_r[...] + u * jax.nn.sigmoid(u)
        t = _dot(hn, wo1_r[...])
        t = t * jax.nn.sigmoid(t)
        lane = lax.broadcasted_iota(jnp.int32, (BN, 16), 1)
        ones_col = jnp.where(lane == 1, 1.0, 0.0).astype(_f32)
        node_r[...] = _dot(t, wo2_r[...]) + ones_col

    return pl.pallas_call(
        body,
        grid=(NPAD // BN,),
        in_specs=[pl.BlockSpec((BN, H), lambda i: (i, 0))] * 3
        + [pl.BlockSpec((H, H), lambda i: (0, 0)),
           pl.BlockSpec((H, 64), lambda i: (0, 0)),
           pl.BlockSpec((64, 16), lambda i: (0, 0))],
        out_specs=pl.BlockSpec((BN, 16), lambda i: (i, 0)),
        out_shape=jax.ShapeDtypeStruct((NPAD, 16), _f32),
    )(h, a0, a1, wu, wo1, wo2p)


def _tc_finalize(pool):
    def body(p_r, o_r):
        s = jnp.sum(p_r[...], axis=0)
        r16 = lax.broadcasted_iota(jnp.int32, (16, 16), 0)
        sel0 = jnp.where(r16 == 0, 1.0, 0.0).astype(_f32)
        sel1 = jnp.where(r16 == 1, 1.0, 0.0).astype(_f32)
        o_r[...] = _dot(s, sel0) / jnp.maximum(_dot(s, sel1), 1.0)

    return pl.pallas_call(
        body, out_shape=jax.ShapeDtypeStruct((GR, 16), _f32),
    )(pool)


def kernel(z, pos, edge_index, batch, comp_emb, period_emb, group_emb,
           phys_table, W_phys, W_edge, W_msg, W_upd, W_out1, W_out2):
    srcf = edge_index[0].astype(jnp.int32)
    dstf = edge_index[1].astype(jnp.int32)
    src3 = srcf.reshape(NW, NCH_E, CCH)
    dst3 = dstf.reshape(NW, NCH_E, CCH)
    z_pad = jnp.concatenate([z.astype(jnp.int32),
                             jnp.zeros((NPAD - N,), jnp.int32)])
    batch_pad = jnp.concatenate([batch.astype(jnp.int32),
                                 jnp.full((NPAD - N,), NGRAPH, jnp.int32)])
    posp = jnp.pad(pos.astype(_f32), ((0, 0), (0, 13)))
    ce_pad = jnp.pad(comp_emb, ((0, 28), (0, 0)))
    pt_pad = jnp.pad(phys_table, ((0, 28), (0, 0)))
    we_rbf = jnp.pad(W_edge[:NG], ((0, 64 - NG), (0, 0)))
    we_rel = jnp.pad(W_edge[NG:NG + 3], ((0, 13), (0, 0)))
    wtop = W_msg[:, :H, :]
    wbot = W_msg[:, H:, :]
    wo2p = jnp.pad(W_out2, ((0, 0), (0, 15)))
    zeros_n = jnp.zeros((NPAD, H), _f32)
    zeros_g = jnp.zeros((GR, 16), _f32)

    htab, ptab = _tc_htab(ce_pad, pt_pad, W_phys, period_emb, group_emb,
                          wtop[0])
    h, p = _sc_gather2(htab, ptab, z_pad)
    rel = _sc_rel(posp, src3, dst3)
    ep = _tc_edge(rel, we_rbf, we_rel, wbot)

    for i in range(4):
        aggp = _sc_msg(p, ep[i], srcf, dstf, zeros_n)
        if i < 3:
            h, p = _tc_update(h, aggp[0], aggp[1], W_upd[i], wtop[i + 1])
        else:
            node = _tc_final_node(h, aggp[0], aggp[1], W_upd[3], W_out1, wo2p)

    pool = _sc_pool(node, batch_pad, zeros_g)
    res = _tc_finalize(pool)
    return res[:NGRAPH, 0:1]
